# Initial kernel scaffold; baseline (speedup 1.0000x reference)
import jax
import jax.numpy as jnp
from jax.experimental import pallas as pl


def kernel(token_ids, weight):
    raise NotImplementedError("write your pallas kernel here")


# speedup vs baseline: 4.8074x; 4.8074x over previous
import functools

import jax
import jax.numpy as jnp
from jax import lax
from jax.experimental import pallas as pl
from jax.experimental.pallas import tpu as pltpu
from jax.experimental.pallas import tpu_sc as plsc

NUM_EMB = 1000000
DIM = 32
NC = 2
NS = 16
NW = NC * NS


def _make_lookup(n_tokens: int, chunk: int):
  assert n_tokens % (NW * chunk) == 0
  per_w = n_tokens // NW
  n_chunks = per_w // chunk
  mesh = plsc.VectorSubcoreMesh(core_axis_name="c", subcore_axis_name="s")

  @functools.partial(
      pl.kernel,
      mesh=mesh,
      out_type=jax.ShapeDtypeStruct((n_tokens, DIM), jnp.float32),
      compiler_params=pltpu.CompilerParams(use_tc_tiling_on_sc=False),
      scratch_types=[
          pltpu.VMEM((chunk,), jnp.int32),
          pltpu.VMEM((chunk, DIM), jnp.float32),
          pltpu.SemaphoreType.DMA,
      ],
  )
  def lookup(idx_hbm, table_hbm, out_hbm, idx_v, rows_v, sem):
    wid = lax.axis_index("s") * NC + lax.axis_index("c")
    base = wid * per_w

    def body(g, carry):
      off = base + g * chunk
      pltpu.sync_copy(idx_hbm.at[pl.ds(off, chunk)], idx_v)
      pltpu.async_copy(table_hbm.at[idx_v], rows_v, sem).wait()
      pltpu.sync_copy(rows_v, out_hbm.at[pl.ds(off, chunk)])
      return carry

    lax.fori_loop(0, n_chunks, body, 0)

  return lookup


def kernel(token_ids, weight):
  b, t = token_ids.shape
  n_tokens = b * t
  idx_flat = token_ids.reshape(n_tokens).astype(jnp.int32)
  out = _make_lookup(n_tokens, 1024)(idx_flat, weight)
  return out.reshape(b, t, DIM)


# speedup vs baseline: 5.0389x; 1.0482x over previous
import functools

import jax
import jax.numpy as jnp
from jax import lax
from jax.experimental import pallas as pl
from jax.experimental.pallas import tpu as pltpu
from jax.experimental.pallas import tpu_sc as plsc

NUM_EMB = 1000000
DIM = 32
NC = 2
NS = 16
NW = NC * NS
CHUNK = 800
NBUF = 4


def _make_lookup(n_tokens: int):
  per_w = n_tokens // NW
  n_chunks = per_w // CHUNK
  assert per_w % CHUNK == 0 and n_chunks % NBUF == 0 and CHUNK % 8 == 0
  mesh = plsc.VectorSubcoreMesh(core_axis_name="c", subcore_axis_name="s")

  @functools.partial(
      pl.kernel,
      mesh=mesh,
      out_type=jax.ShapeDtypeStruct((n_tokens, DIM), jnp.float32),
      compiler_params=pltpu.CompilerParams(use_tc_tiling_on_sc=False),
      scratch_types=[
          pltpu.VMEM((NBUF, CHUNK), jnp.int32),
          pltpu.VMEM((NBUF, CHUNK, DIM), jnp.float32),
          [pltpu.SemaphoreType.DMA] * NBUF,
          [pltpu.SemaphoreType.DMA] * NBUF,
          [pltpu.SemaphoreType.DMA] * NBUF,
      ],
  )
  def lookup(idx_hbm, table_hbm, out_hbm, idx_v, rows_v, sidx, sgat, sout):
    wid = lax.axis_index("s") * NC + lax.axis_index("c")
    base = wid * per_w

    def idx_copy(g, j):
      off = base + g * CHUNK
      return pltpu.make_async_copy(
          idx_hbm.at[pl.ds(off, CHUNK)], idx_v.at[j], sidx[j])

    def gather_copy(j):
      return pltpu.make_async_copy(
          table_hbm.at[idx_v.at[j]], rows_v.at[j], sgat[j])

    def out_copy(g, j):
      off = base + g * CHUNK
      return pltpu.make_async_copy(
          rows_v.at[j], out_hbm.at[pl.ds(off, CHUNK)], sout[j])

    for j in range(NBUF):
      idx_copy(j, j).start()

    def group(g0, carry):
      for j in range(NBUF):
        g = g0 + j
        pl.when(g0 > 0)(lambda g=g, j=j: out_copy(g - NBUF, j).wait())
        idx_copy(g, j).wait()
        gather_copy(j).start()
      for j in range(NBUF):
        g = g0 + j
        gather_copy(j).wait()
        out_copy(g, j).start()
        pl.when(g + NBUF < n_chunks)(lambda g=g, j=j: idx_copy(g + NBUF, j).start())
      return carry

    lax.fori_loop(0, n_chunks // NBUF, lambda i, c: group(i * NBUF, c), 0,
                  unroll=False)

    for j in range(NBUF):
      out_copy(n_chunks - NBUF + j, j).wait()

  return lookup


def kernel(token_ids, weight):
  b, t = token_ids.shape
  n_tokens = b * t
  idx_flat = token_ids.reshape(n_tokens).astype(jnp.int32)
  out = _make_lookup(n_tokens)(idx_flat, weight)
  return out.reshape(b, t, DIM)
